# baseline (speedup 1.0000x reference)
import jax
import jax.numpy as jnp
from jax import lax
from jax.experimental import pallas as pl
from jax.experimental.pallas import tpu as pltpu

_H_UPPER = 10.0
_B = 128
_N = 32768
_BN = 16384
_NB = _N // _BN


def _body(h_ref, hm_ref, hv_ref, out_ref,
          hbuf, mnew_s, vnew_s, colneg_s, svar, rmax, ridx):
    p = pl.program_id(0)
    j = pl.program_id(1)
    ds = pl.ds(j * _BN, _BN)

    @pl.when(p == 0)
    def _phase0():
        xb = h_ref[...]
        hbuf[:, ds] = xb
        ones = jnp.full((1, _B), 1.0 / _B, jnp.float32)
        mu = jnp.dot(ones, xb, preferred_element_type=jnp.float32)
        msq = jnp.dot(ones, xb * xb, preferred_element_type=jnp.float32)
        var = msq - mu * mu
        hm = hm_ref[...]
        hv = hv_ref[...]
        mn = (hm * _H_UPPER + mu) / (_H_UPPER + 1.0)
        vn = (hv * (_H_UPPER - 1.0 / _B) + var
              + (mu - hm) ** 2 / (1.0 + 1.0 / _H_UPPER)) \
            / (_H_UPPER + 1.0 - 1.0 / _B)
        mnew_s[:, ds] = mn
        vnew_s[:, ds] = vn

        @pl.when(j == 0)
        def _():
            svar[0, 0] = 0.0
            colneg_s[...] = -lax.broadcasted_iota(
                jnp.int32, (1, _BN), 1).astype(jnp.float32)
        svar[0, 0] += jnp.sum(vn)

    @pl.when(p == 1)
    def _phase1():
        xb = hbuf[:, ds]
        mb = mnew_s[:, ds]
        vb = vnew_s[:, ds]
        c = svar[0, 0] / (float(_N) * 100.0)
        rinv = 1.0 / (vb + c)
        d = xb - mb
        score = d * d * rinv
        bmax = jnp.max(score, axis=1, keepdims=True)
        cn = colneg_s[...] - (j * _BN).astype(jnp.float32)
        cand = jnp.where(score == bmax, cn, -jnp.inf)
        barg = jnp.max(cand, axis=1, keepdims=True)

        @pl.when(j == 0)
        def _():
            rmax[...] = bmax
            ridx[...] = barg

        @pl.when(j != 0)
        def _():
            better = bmax > rmax[...]
            rmax[...] = jnp.where(better, bmax, rmax[...])
            ridx[...] = jnp.where(better, barg, ridx[...])

    @pl.when(p == 2)
    def _phase2():
        xb = hbuf[:, ds]
        mb = mnew_s[:, ds]
        cn = colneg_s[...] - (j * _BN).astype(jnp.float32)
        sel = cn == ridx[...]
        out_ref[...] = jnp.where(sel, jnp.broadcast_to(mb, xb.shape), xb)


def _build(interpret):
    return pl.pallas_call(
        _body,
        grid=(3, _NB),
        in_specs=[
            pl.BlockSpec((_B, _BN), lambda p, j: (0, jnp.where(p == 0, j, 0))),
            pl.BlockSpec((1, _BN), lambda p, j: (0, jnp.where(p == 0, j, 0))),
            pl.BlockSpec((1, _BN), lambda p, j: (0, jnp.where(p == 0, j, 0))),
        ],
        out_specs=pl.BlockSpec((_B, _BN), lambda p, j: (0, jnp.where(p == 2, j, 0))),
        out_shape=jax.ShapeDtypeStruct((_B, _N), jnp.float32),
        scratch_shapes=[
            pltpu.VMEM((_B, _N), jnp.float32),
            pltpu.VMEM((1, _N), jnp.float32),
            pltpu.VMEM((1, _N), jnp.float32),
            pltpu.VMEM((1, _BN), jnp.float32),
            pltpu.SMEM((1, 1), jnp.float32),
            pltpu.VMEM((_B, 1), jnp.float32),
            pltpu.VMEM((_B, 1), jnp.float32),
        ],
        compiler_params=pltpu.CompilerParams(
            dimension_semantics=("arbitrary", "arbitrary"),
        ),
        interpret=interpret,
    )


@jax.jit
def kernel(h, h_mean, h_var):
    return _build(False)(h, h_mean, h_var)
